# baseline (speedup 1.0000x reference)
import jax
import jax.numpy as jnp
from jax import lax
from jax.experimental import pallas as pl
from jax.experimental.pallas import tpu as pltpu
from jax.experimental.pallas import tpu_sc as plsc

NU = 50000
NI = 50000
D = 32
K = 16
E = 800000
NLAYERS = 2

NS = 16
L = 16

SPAN_BIG = 3200
SPAN_LAST = NU - 15 * SPAN_BIG

_MESH = plsc.VectorSubcoreMesh(core_axis_name="c", subcore_axis_name="s")
_SC_PARAMS = pltpu.CompilerParams(use_tc_tiling_on_sc=False)


def _sds(shape, dtype=jnp.float32):
    return jax.ShapeDtypeStruct(shape, dtype)


def _zero_bins(zeros_hbm, bins_sh, sid):
    start = sid * SPAN_BIG

    @pl.when(sid < 15)
    def _():
        pltpu.sync_copy(zeros_hbm.at[pl.ds(start, SPAN_BIG), :],
                        bins_sh.at[pl.ds(start, SPAN_BIG), :])

    @pl.when(sid == 15)
    def _():
        pltpu.sync_copy(zeros_hbm.at[pl.ds(start, SPAN_LAST), :],
                        bins_sh.at[pl.ds(start, SPAN_LAST), :])


def _flush_bins(bins_sh, out_hbm, sid):
    start = sid * SPAN_BIG

    @pl.when(sid < 15)
    def _():
        pltpu.sync_copy(bins_sh.at[pl.ds(start, SPAN_BIG), :],
                        out_hbm.at[pl.ds(start, SPAN_BIG), :])

    @pl.when(sid == 15)
    def _():
        pltpu.sync_copy(bins_sh.at[pl.ds(start, SPAN_LAST), :],
                        out_hbm.at[pl.ds(start, SPAN_LAST), :])


def _pipe_gather_scatter(dst_hbm, src_hbm, table, bins_sh,
                         idx_d, idx_s, rows, sems,
                         base0, nch, chunk,
                         dat_hbm=None, data_v=None, scale_fn=None):
    def loadidx(k, b):
        base = base0 + k * chunk
        pltpu.sync_copy(dst_hbm.at[pl.ds(base, chunk)], idx_d[b])
        pltpu.sync_copy(src_hbm.at[pl.ds(base, chunk)], idx_s[b])
        if dat_hbm is not None:
            pltpu.sync_copy(dat_hbm.at[pl.ds(base, chunk)], data_v[b])

    def startg(b):
        pltpu.async_copy(table.at[idx_s[b]], rows[b], sems[b])

    def waitg(b):
        pltpu.make_async_copy(table.at[idx_s[b]], rows[b], sems[b]).wait()

    def scat(b):
        if scale_fn is not None:
            scale_fn(rows[b], data_v[b])
        pltpu.sync_copy(rows[b], bins_sh.at[idx_d[b]], add=True)

    loadidx(0, 0)
    startg(0)

    def body(m, _):
        k1 = 2 * m + 1
        loadidx(k1, 1)
        startg(1)
        waitg(0)
        scat(0)
        loadidx(k1 + 1, 0)
        startg(0)
        waitg(1)
        scat(1)
        return 0
    lax.fori_loop(0, (nch - 1) // 2, body, 0)
    waitg(0)
    scat(0)


def _rsqrt16(x):
    i = lax.bitcast_convert_type(x, jnp.int32)
    i = jnp.int32(0x5F3759DF) - jnp.right_shift(i, 1)
    y = lax.bitcast_convert_type(i, jnp.float32)
    for _ in range(3):
        y = y * (1.5 - 0.5 * x * y * y)
    return jnp.where(x > 0.0, y, 0.0)


def _recip16(x):
    return jnp.where(x > 0.0, 1.0 / x, 0.0)


def _precompute_body(h_hbm, t_hbm, uuh_hbm, uud_hbm, iih_hbm, iid_hbm,
                     zeros_hbm,
                     v_u_out, v_i_out, inv_u_out, inv_i_out,
                     idx_v, data_v, ones_v, upd_v, binbuf_v,
                     bins_deg, bins_sum):
    cid = lax.axis_index("c")
    sid = lax.axis_index("s")
    iota = lax.iota(jnp.int32, L)
    zero16 = iota * 0

    _zero_bins(zeros_hbm, bins_deg, sid)
    _zero_bins(zeros_hbm, bins_sum, sid)

    ones = (zero16 + 1).astype(jnp.float32)

    def fill(g, _):
        ones_v[g, :] = ones
        return 0
    lax.fori_loop(0, 400, fill, 0)
    plsc.subcore_barrier()

    def deg_chunk(k, deg_src):
        base = sid * 50000 + k * 400
        pltpu.sync_copy(deg_src.at[pl.ds(base, 400)], idx_v)
        pltpu.sync_copy(ones_v, bins_deg.at[idx_v], add=True)
        return 0

    @pl.when(cid == 0)
    def _():
        lax.fori_loop(0, 125, lambda k, c: deg_chunk(k, h_hbm), 0)

    @pl.when(cid == 1)
    def _():
        lax.fori_loop(0, 125, lambda k, c: deg_chunk(k, t_hbm), 0)

    def sum_loop(src_idx, src_dat):
        sbase = sid * 26000
        nch = jnp.where(sid < 15, 65, 25)

        def body(k, _):
            base = sbase + k * 400
            pltpu.sync_copy(src_idx.at[pl.ds(base, 400)], idx_v)
            pltpu.sync_copy(src_dat.at[pl.ds(base, 400)], data_v)

            def put(g, _):
                d16 = data_v[pl.ds(g * L, L)]
                for j in range(L):
                    r = g * L + j
                    upd_v[r, :] = ones_v[r, :] * d16[j]
                return 0
            lax.fori_loop(0, 400 // L, put, 0)
            pltpu.sync_copy(upd_v, bins_sum.at[idx_v], add=True)
            return 0
        lax.fori_loop(0, nch, body, 0)

    @pl.when(cid == 0)
    def _():
        sum_loop(uuh_hbm, uud_hbm)

    @pl.when(cid == 1)
    def _():
        sum_loop(iih_hbm, iid_hbm)

    plsc.subcore_barrier()

    def finalize(bins, out_hbm, fn, nsl):
        def sl(m, _):
            start = sid * SPAN_BIG + m * 400
            pltpu.sync_copy(bins.at[pl.ds(start, 400), :], binbuf_v)

            def body(r, _):
                binbuf_v[r, :] = fn(binbuf_v[r, :])
                return 0
            lax.fori_loop(0, 400, body, 0)
            pltpu.sync_copy(binbuf_v, out_hbm.at[pl.ds(start, 400), :])
            return 0
        lax.fori_loop(0, nsl, sl, 0)

    nsl = jnp.where(sid < 15, SPAN_BIG // 400, SPAN_LAST // 400)

    @pl.when(cid == 0)
    def _():
        finalize(bins_deg, v_u_out, _rsqrt16, nsl)
        finalize(bins_sum, inv_u_out, _recip16, nsl)

    @pl.when(cid == 1)
    def _():
        finalize(bins_deg, v_i_out, _rsqrt16, nsl)
        finalize(bins_sum, inv_i_out, _recip16, nsl)


def _precompute_sc(h, t, uuh, uud, iih, iid, zeros16):
    return pl.kernel(
        _precompute_body,
        out_type=(_sds((NU, L)), _sds((NI, L)), _sds((NU, L)), _sds((NI, L))),
        mesh=_MESH,
        compiler_params=_SC_PARAMS,
        scratch_types=[
            pltpu.VMEM((400,), jnp.int32),
            pltpu.VMEM((400,), jnp.float32),
            pltpu.VMEM((400, L), jnp.float32),
            pltpu.VMEM((400, L), jnp.float32),
            pltpu.VMEM((400, L), jnp.float32),
            pltpu.VMEM_SHARED((NU, L), jnp.float32),
            pltpu.VMEM_SHARED((NU, L), jnp.float32),
        ],
    )(h, t, uuh, uud, iih, iid, zeros16)


def _gnnside_body(ego_s_u, ego_s_i, ego_u, ego_i,
                  h_hbm, t_hbm, uuh, uut, uud, iih, iit, iid, zeros_hbm,
                  binsU_out, binsI_out, sideU_out, sideI_out,
                  idx_d0, idx_d1, idx_s0, idx_s1, rows0, rows1,
                  data0, data1, sem0, sem1, bins_sh):
    cid = lax.axis_index("c")
    sid = lax.axis_index("s")
    idx_d = [idx_d0, idx_d1]
    idx_s = [idx_s0, idx_s1]
    rows = [rows0, rows1]
    data = [data0, data1]
    sems = [sem0, sem1]

    _zero_bins(zeros_hbm, bins_sh, sid)
    plsc.subcore_barrier()

    def runA(dst_hbm, src_hbm, table):
        _pipe_gather_scatter(dst_hbm, src_hbm, table, bins_sh,
                             idx_d, idx_s, rows, sems,
                             sid * 50000, 125, 400)

    @pl.when(cid == 0)
    def _():
        runA(h_hbm, t_hbm, ego_s_i)

    @pl.when(cid == 1)
    def _():
        runA(t_hbm, h_hbm, ego_s_u)

    plsc.subcore_barrier()

    @pl.when(cid == 0)
    def _():
        _flush_bins(bins_sh, binsU_out, sid)

    @pl.when(cid == 1)
    def _():
        _flush_bins(bins_sh, binsI_out, sid)

    _zero_bins(zeros_hbm, bins_sh, sid)
    plsc.subcore_barrier()

    def scale_rows(rbuf, dbuf):
        def scale(g, _):
            d16 = dbuf[pl.ds(g * L, L)]
            for j in range(L):
                r = g * L + j
                rbuf[r, pl.ds(0, L)] = rbuf[r, pl.ds(0, L)] * d16[j]
                rbuf[r, pl.ds(L, L)] = rbuf[r, pl.ds(L, L)] * d16[j]
            return 0
        lax.fori_loop(0, 400 // L, scale, 0)

    def runB(dst_hbm, src_hbm, dat_hbm, table):
        nch = jnp.where(sid < 15, 65, 25)
        _pipe_gather_scatter(dst_hbm, src_hbm, table, bins_sh,
                             idx_d, idx_s, rows, sems,
                             sid * 26000, nch, 400,
                             dat_hbm=dat_hbm, data_v=data,
                             scale_fn=scale_rows)

    @pl.when(cid == 0)
    def _():
        runB(uuh, uut, uud, ego_u)

    @pl.when(cid == 1)
    def _():
        runB(iih, iit, iid, ego_i)

    plsc.subcore_barrier()

    @pl.when(cid == 0)
    def _():
        _flush_bins(bins_sh, sideU_out, sid)

    @pl.when(cid == 1)
    def _():
        _flush_bins(bins_sh, sideI_out, sid)


def _gnnside_sc(ego_s_u, ego_s_i, ego_u, ego_i,
                h, t, uuh, uut, uud, iih, iit, iid, zeros32):
    return pl.kernel(
        _gnnside_body,
        out_type=(_sds((NU, D)), _sds((NI, D)), _sds((NU, D)), _sds((NI, D))),
        mesh=_MESH,
        compiler_params=_SC_PARAMS,
        scratch_types=[
            pltpu.VMEM((400,), jnp.int32),
            pltpu.VMEM((400,), jnp.int32),
            pltpu.VMEM((400,), jnp.int32),
            pltpu.VMEM((400,), jnp.int32),
            pltpu.VMEM((400, D), jnp.float32),
            pltpu.VMEM((400, D), jnp.float32),
            pltpu.VMEM((400,), jnp.float32),
            pltpu.VMEM((400,), jnp.float32),
            pltpu.SemaphoreType.DMA,
            pltpu.SemaphoreType.DMA,
            pltpu.VMEM_SHARED((NU, D), jnp.float32),
        ],
    )(ego_s_u, ego_s_i, ego_u, ego_i, h, t, uuh, uut, uud, iih, iit, iid,
      zeros32)


def _pmul_body(gnn_u, gnn_i, h_hbm, t_hbm, p_out,
               idx_h, idx_t, rh0, rh1, rt0, rt1, semh0, semh1, semt0, semt1):
    cid = lax.axis_index("c")
    sid = lax.axis_index("s")
    rows_h = [rh0, rh1]
    rows_t = [rt0, rt1]
    sems_h = [semh0, semh1]
    sems_t = [semt0, semt1]
    base0 = cid * 400000 + sid * 25000
    C = 1000

    def loadidx(k):
        base = base0 + k * C
        pltpu.sync_copy(h_hbm.at[pl.ds(base, C)], idx_h)
        pltpu.sync_copy(t_hbm.at[pl.ds(base, C)], idx_t)

    def startg(b):
        pltpu.async_copy(gnn_u.at[idx_h], rows_h[b], sems_h[b])
        pltpu.async_copy(gnn_i.at[idx_t], rows_t[b], sems_t[b])

    def waitg(b):
        pltpu.make_async_copy(gnn_u.at[idx_h], rows_h[b], sems_h[b]).wait()
        pltpu.make_async_copy(gnn_i.at[idx_t], rows_t[b], sems_t[b]).wait()

    def muland(b, k):
        def mul(r, _):
            rows_h[b][r, :] = rows_h[b][r, :] * rows_t[b][r, :]
            return 0
        lax.fori_loop(0, C, mul, 0)
        pltpu.sync_copy(rows_h[b], p_out.at[pl.ds(base0 + k * C, C), :])

    loadidx(0)
    startg(0)

    def body(m, _):
        k1 = 2 * m + 1
        waitg(0)
        loadidx(k1)
        startg(1)
        muland(0, k1 - 1)
        waitg(1)
        loadidx(k1 + 1)
        startg(0)
        muland(1, k1)
        return 0
    lax.fori_loop(0, 12, body, 0)
    waitg(0)
    muland(0, 24)


def _pmul_sc(gnnb_u, gnnb_i, h, t):
    return pl.kernel(
        _pmul_body,
        out_type=_sds((E, D), jnp.bfloat16),
        mesh=_MESH,
        compiler_params=_SC_PARAMS,
        scratch_types=[
            pltpu.VMEM((1000,), jnp.int32),
            pltpu.VMEM((1000,), jnp.int32),
            pltpu.VMEM((1000, D), jnp.bfloat16),
            pltpu.VMEM((1000, D), jnp.bfloat16),
            pltpu.VMEM((1000, D), jnp.bfloat16),
            pltpu.VMEM((1000, D), jnp.bfloat16),
            pltpu.SemaphoreType.DMA,
            pltpu.SemaphoreType.DMA,
            pltpu.SemaphoreType.DMA,
            pltpu.SemaphoreType.DMA,
        ],
    )(gnnb_u, gnnb_i, h, t)


def _sd_scatter_body(sd_hbm, h_hbm, t_hbm, zeros_hbm,
                     sU_out, sI_out,
                     idx_d0, idx_d1, rows0, rows1, sem0, sem1, bins_sh):
    cid = lax.axis_index("c")
    sid = lax.axis_index("s")
    idx_d = [idx_d0, idx_d1]
    rows = [rows0, rows1]
    sems = [sem0, sem1]
    _zero_bins(zeros_hbm, bins_sh, sid)
    plsc.subcore_barrier()
    C = 2000

    def run(dst_hbm):
        def loadc(k, b):
            base = sid * 50000 + k * C
            pltpu.sync_copy(dst_hbm.at[pl.ds(base, C)], idx_d[b])
            pltpu.async_copy(sd_hbm.at[pl.ds(base, C), :], rows[b], sems[b])

        def waitc(k, b):
            base = sid * 50000 + k * C
            pltpu.make_async_copy(sd_hbm.at[pl.ds(base, C), :], rows[b],
                                  sems[b]).wait()

        def scat(b):
            pltpu.sync_copy(rows[b], bins_sh.at[idx_d[b]], add=True)

        loadc(0, 0)

        def body(m, _):
            k1 = 2 * m + 1
            loadc(k1, 1)
            waitc(k1 - 1, 0)
            scat(0)
            loadc(k1 + 1, 0)
            waitc(k1, 1)
            scat(1)
            return 0
        lax.fori_loop(0, 12, body, 0)
        waitc(24, 0)
        scat(0)

    @pl.when(cid == 0)
    def _():
        run(h_hbm)

    @pl.when(cid == 1)
    def _():
        run(t_hbm)

    plsc.subcore_barrier()

    @pl.when(cid == 0)
    def _():
        _flush_bins(bins_sh, sU_out, sid)

    @pl.when(cid == 1)
    def _():
        _flush_bins(bins_sh, sI_out, sid)


def _sd_scatter_sc(sd, h, t, zeros16):
    return pl.kernel(
        _sd_scatter_body,
        out_type=(_sds((NU, K)), _sds((NI, K))),
        mesh=_MESH,
        compiler_params=_SC_PARAMS,
        scratch_types=[
            pltpu.VMEM((2000,), jnp.int32),
            pltpu.VMEM((2000,), jnp.int32),
            pltpu.VMEM((2000, K), jnp.float32),
            pltpu.VMEM((2000, K), jnp.float32),
            pltpu.SemaphoreType.DMA,
            pltpu.SemaphoreType.DMA,
            pltpu.VMEM_SHARED((NU, K), jnp.float32),
        ],
    )(sd, h, t, zeros16)


_TBLK = 2000


def _scale_body(x_ref, v_ref, o_ref):
    o_ref[...] = x_ref[...] * v_ref[...]


def _scale2_body(xu_ref, xi_ref, vu_ref, vi_ref,
                 ou_ref, oi_ref, obu_ref, obi_ref):
    gu = xu_ref[...] * vu_ref[...]
    gi = xi_ref[...] * vi_ref[...]
    ou_ref[...] = gu
    oi_ref[...] = gi
    obu_ref[...] = gu.astype(jnp.bfloat16)
    obi_ref[...] = gi.astype(jnp.bfloat16)


def _scale2_tc(xu, xi, vu2, vi2):
    blkd = pl.BlockSpec((_TBLK, D), lambda i: (i, 0))
    blk1 = pl.BlockSpec((_TBLK, 1), lambda i: (i, 0))
    return pl.pallas_call(
        _scale2_body,
        grid=(NU // _TBLK,),
        in_specs=[blkd, blkd, blk1, blk1],
        out_specs=[blkd, blkd, blkd, blkd],
        out_shape=(_sds((NU, D)), _sds((NI, D)),
                   _sds((NU, D), jnp.bfloat16), _sds((NI, D), jnp.bfloat16)),
    )(xu, xi, vu2, vi2)


def _scale_tc(x, v2d):
    n = x.shape[0]
    return pl.pallas_call(
        _scale_body,
        grid=(n // _TBLK,),
        in_specs=[pl.BlockSpec((_TBLK, x.shape[1]), lambda i: (i, 0)),
                  pl.BlockSpec((_TBLK, 1), lambda i: (i, 0))],
        out_specs=pl.BlockSpec((_TBLK, x.shape[1]), lambda i: (i, 0)),
        out_shape=_sds((n, x.shape[1])),
    )(x, v2d)


def _intent_body(p_ref, w_ref, o_ref):
    p = p_ref[...].astype(jnp.float32)
    logits = jnp.dot(p, w_ref[...], preferred_element_type=jnp.float32)
    m = jnp.max(logits, axis=1, keepdims=True)
    e = jnp.exp(logits - m)
    dist = e / jnp.sum(e, axis=1, keepdims=True)
    alpha = (jnp.sum(p, axis=1, keepdims=True) + 1.0) * 0.5
    o_ref[...] = alpha * dist


def _intent_tc(p, w):
    blk = 8000
    return pl.pallas_call(
        _intent_body,
        grid=(E // blk,),
        in_specs=[pl.BlockSpec((blk, D), lambda i: (i, 0)),
                  pl.BlockSpec((D, K), lambda i: (0, 0))],
        out_specs=pl.BlockSpec((blk, K), lambda i: (i, 0)),
        out_shape=_sds((E, K)),
    )(p, w)


def _combine_body(gnn_ref, side_ref, s_ref, w_ref, inv_ref, v_ref, ae_ref,
                  ego_ref, egos_ref, aeo_ref):
    intent = lax.dot_general(s_ref[...], w_ref[...],
                             (((1,), (1,)), ((), ())),
                             preferred_element_type=jnp.float32)
    ego = gnn_ref[...] + inv_ref[...] * side_ref[...] + intent
    ego_ref[...] = ego
    egos_ref[...] = v_ref[...] * ego
    aeo_ref[...] = ae_ref[...] + ego


def _combine2_body(gu_ref, su_ref, squ_ref, w_ref, iu_ref, vu_ref, au_ref,
                   gi_ref, si_ref, sqi_ref, ii_ref, vi_ref, ai_ref,
                   egou_ref, egosu_ref, aeu_ref,
                   egoi_ref, egosi_ref, aei_ref):
    dn = (((1,), (1,)), ((), ()))
    intu = lax.dot_general(squ_ref[...], w_ref[...], dn,
                           preferred_element_type=jnp.float32)
    inti = lax.dot_general(sqi_ref[...], w_ref[...], dn,
                           preferred_element_type=jnp.float32)
    egou = gu_ref[...] + iu_ref[...] * su_ref[...] + intu
    egoi = gi_ref[...] + ii_ref[...] * si_ref[...] + inti
    egou_ref[...] = egou
    egoi_ref[...] = egoi
    egosu_ref[...] = vu_ref[...] * egou
    egosi_ref[...] = vi_ref[...] * egoi
    aeu_ref[...] = au_ref[...] + egou
    aei_ref[...] = ai_ref[...] + egoi


def _combine2_tc(gnn_u, sideraw_u, s_u, w, inv_u2, v_u2, ae_u,
                 gnn_i, sideraw_i, s_i, inv_i2, v_i2, ae_i):
    blkd = pl.BlockSpec((_TBLK, D), lambda i: (i, 0))
    blkk = pl.BlockSpec((_TBLK, K), lambda i: (i, 0))
    blk1 = pl.BlockSpec((_TBLK, 1), lambda i: (i, 0))
    blkw = pl.BlockSpec((D, K), lambda i: (0, 0))
    return pl.pallas_call(
        _combine2_body,
        grid=(NU // _TBLK,),
        in_specs=[blkd, blkd, blkk, blkw, blk1, blk1, blkd,
                  blkd, blkd, blkk, blk1, blk1, blkd],
        out_specs=[blkd, blkd, blkd, blkd, blkd, blkd],
        out_shape=(_sds((NU, D)), _sds((NU, D)), _sds((NU, D)),
                   _sds((NI, D)), _sds((NI, D)), _sds((NI, D))),
    )(gnn_u, sideraw_u, s_u, w, inv_u2, v_u2, ae_u,
      gnn_i, sideraw_i, s_i, inv_i2, v_i2, ae_i)


def kernel(user_emb, item_emb, intents, uu_data, ii_data,
           h_list, t_list, uu_h, uu_t, ii_h, ii_t):
    zeros32 = jnp.zeros((NU, D), jnp.float32)
    zeros16 = jnp.zeros((NU, K), jnp.float32)

    v_u, v_i, inv_u, inv_i = _precompute_sc(h_list, t_list, uu_h, uu_data,
                                            ii_h, ii_data, zeros16)
    v_u2 = v_u[:, 0:1]
    v_i2 = v_i[:, 0:1]
    inv_u2 = inv_u[:, 0:1]
    inv_i2 = inv_i[:, 0:1]

    ego_u, ego_i = user_emb, item_emb
    ae_u, ae_i = user_emb, item_emb
    ego_s_u = _scale_tc(ego_u, v_u2)
    ego_s_i = _scale_tc(ego_i, v_i2)

    for _ in range(NLAYERS):
        binsA_u, binsA_i, sideraw_u, sideraw_i = _gnnside_sc(
            ego_s_u, ego_s_i, ego_u, ego_i, h_list, t_list,
            uu_h, uu_t, uu_data, ii_h, ii_t, ii_data, zeros32)
        gnn_u, gnn_i, gnnb_u, gnnb_i = _scale2_tc(binsA_u, binsA_i,
                                                  v_u2, v_i2)
        p = _pmul_sc(gnnb_u, gnnb_i, h_list, t_list)
        sd = _intent_tc(p, intents)
        s_u, s_i = _sd_scatter_sc(sd, h_list, t_list, zeros16)
        (ego_u, ego_s_u, ae_u,
         ego_i, ego_s_i, ae_i) = _combine2_tc(
            gnn_u, sideraw_u, s_u, intents, inv_u2, v_u2, ae_u,
            gnn_i, sideraw_i, s_i, inv_i2, v_i2, ae_i)

    return jnp.concatenate([ae_u, ae_i], axis=0)


# speedup vs baseline: 1.0382x; 1.0382x over previous
import jax
import jax.numpy as jnp
from jax import lax
from jax.experimental import pallas as pl
from jax.experimental.pallas import tpu as pltpu
from jax.experimental.pallas import tpu_sc as plsc

NU = 50000
NI = 50000
D = 32
K = 16
E = 800000
NLAYERS = 2

NS = 16
L = 16

SPAN_BIG = 3200
SPAN_LAST = NU - 15 * SPAN_BIG

_MESH = plsc.VectorSubcoreMesh(core_axis_name="c", subcore_axis_name="s")
_SC_PARAMS = pltpu.CompilerParams(use_tc_tiling_on_sc=False)


def _sds(shape, dtype=jnp.float32):
    return jax.ShapeDtypeStruct(shape, dtype)


def _zero_bins(zeros_hbm, bins_sh, sid):
    start = sid * SPAN_BIG

    @pl.when(sid < 15)
    def _():
        pltpu.sync_copy(zeros_hbm.at[pl.ds(start, SPAN_BIG), :],
                        bins_sh.at[pl.ds(start, SPAN_BIG), :])

    @pl.when(sid == 15)
    def _():
        pltpu.sync_copy(zeros_hbm.at[pl.ds(start, SPAN_LAST), :],
                        bins_sh.at[pl.ds(start, SPAN_LAST), :])


def _flush_bins(bins_sh, out_hbm, sid):
    start = sid * SPAN_BIG

    @pl.when(sid < 15)
    def _():
        pltpu.sync_copy(bins_sh.at[pl.ds(start, SPAN_BIG), :],
                        out_hbm.at[pl.ds(start, SPAN_BIG), :])

    @pl.when(sid == 15)
    def _():
        pltpu.sync_copy(bins_sh.at[pl.ds(start, SPAN_LAST), :],
                        out_hbm.at[pl.ds(start, SPAN_LAST), :])


def _pipe_gather_scatter(dst_hbm, src_hbm, table, bins_sh,
                         idx_d, idx_s, rows, sems,
                         base0, nch, chunk,
                         dat_hbm=None, data_v=None, scale_fn=None):
    def loadidx(k, b):
        base = base0 + k * chunk
        pltpu.sync_copy(dst_hbm.at[pl.ds(base, chunk)], idx_d[b])
        pltpu.sync_copy(src_hbm.at[pl.ds(base, chunk)], idx_s[b])
        if dat_hbm is not None:
            pltpu.sync_copy(dat_hbm.at[pl.ds(base, chunk)], data_v[b])

    def startg(b):
        pltpu.async_copy(table.at[idx_s[b]], rows[b], sems[b])

    def waitg(b):
        pltpu.make_async_copy(table.at[idx_s[b]], rows[b], sems[b]).wait()

    def scat(b):
        if scale_fn is not None:
            scale_fn(rows[b], data_v[b])
        pltpu.sync_copy(rows[b], bins_sh.at[idx_d[b]], add=True)

    loadidx(0, 0)
    startg(0)

    def body(m, _):
        k1 = 2 * m + 1
        loadidx(k1, 1)
        startg(1)
        waitg(0)
        scat(0)
        loadidx(k1 + 1, 0)
        startg(0)
        waitg(1)
        scat(1)
        return 0
    lax.fori_loop(0, (nch - 1) // 2, body, 0)
    waitg(0)
    scat(0)


def _rsqrt16(x):
    i = lax.bitcast_convert_type(x, jnp.int32)
    i = jnp.int32(0x5F3759DF) - jnp.right_shift(i, 1)
    y = lax.bitcast_convert_type(i, jnp.float32)
    for _ in range(3):
        y = y * (1.5 - 0.5 * x * y * y)
    return jnp.where(x > 0.0, y, 0.0)


def _recip16(x):
    return jnp.where(x > 0.0, 1.0 / x, 0.0)


def _precompute_body(h_hbm, t_hbm, uuh_hbm, uud_hbm, iih_hbm, iid_hbm,
                     zeros_hbm,
                     v_u_out, v_i_out, inv_u_out, inv_i_out,
                     idx_v, data_v, ones_v, upd_v, binbuf_v,
                     bins_deg, bins_sum):
    cid = lax.axis_index("c")
    sid = lax.axis_index("s")
    iota = lax.iota(jnp.int32, L)
    zero16 = iota * 0

    _zero_bins(zeros_hbm, bins_deg, sid)
    _zero_bins(zeros_hbm, bins_sum, sid)

    ones = (zero16 + 1).astype(jnp.float32)

    def fill(g, _):
        ones_v[g, :] = ones
        return 0
    lax.fori_loop(0, 400, fill, 0)
    plsc.subcore_barrier()

    def deg_chunk(k, deg_src):
        base = sid * 50000 + k * 400
        pltpu.sync_copy(deg_src.at[pl.ds(base, 400)], idx_v)
        pltpu.sync_copy(ones_v, bins_deg.at[idx_v], add=True)
        return 0

    @pl.when(cid == 0)
    def _():
        lax.fori_loop(0, 125, lambda k, c: deg_chunk(k, h_hbm), 0)

    @pl.when(cid == 1)
    def _():
        lax.fori_loop(0, 125, lambda k, c: deg_chunk(k, t_hbm), 0)

    def sum_loop(src_idx, src_dat):
        sbase = sid * 26000
        nch = jnp.where(sid < 15, 65, 25)

        def body(k, _):
            base = sbase + k * 400
            pltpu.sync_copy(src_idx.at[pl.ds(base, 400)], idx_v)
            pltpu.sync_copy(src_dat.at[pl.ds(base, 400)], data_v)

            def put(g, _):
                d16 = data_v[pl.ds(g * L, L)]
                for j in range(L):
                    r = g * L + j
                    upd_v[r, :] = ones_v[r, :] * d16[j]
                return 0
            lax.fori_loop(0, 400 // L, put, 0)
            pltpu.sync_copy(upd_v, bins_sum.at[idx_v], add=True)
            return 0
        lax.fori_loop(0, nch, body, 0)

    @pl.when(cid == 0)
    def _():
        sum_loop(uuh_hbm, uud_hbm)

    @pl.when(cid == 1)
    def _():
        sum_loop(iih_hbm, iid_hbm)

    plsc.subcore_barrier()

    def finalize(bins, out_hbm, fn, nsl):
        def sl(m, _):
            start = sid * SPAN_BIG + m * 400
            pltpu.sync_copy(bins.at[pl.ds(start, 400), :], binbuf_v)

            def body(r, _):
                binbuf_v[r, :] = fn(binbuf_v[r, :])
                return 0
            lax.fori_loop(0, 400, body, 0)
            pltpu.sync_copy(binbuf_v, out_hbm.at[pl.ds(start, 400), :])
            return 0
        lax.fori_loop(0, nsl, sl, 0)

    nsl = jnp.where(sid < 15, SPAN_BIG // 400, SPAN_LAST // 400)

    @pl.when(cid == 0)
    def _():
        finalize(bins_deg, v_u_out, _rsqrt16, nsl)
        finalize(bins_sum, inv_u_out, _recip16, nsl)

    @pl.when(cid == 1)
    def _():
        finalize(bins_deg, v_i_out, _rsqrt16, nsl)
        finalize(bins_sum, inv_i_out, _recip16, nsl)


def _precompute_sc(h, t, uuh, uud, iih, iid, zeros16):
    return pl.kernel(
        _precompute_body,
        out_type=(_sds((NU, L)), _sds((NI, L)), _sds((NU, L)), _sds((NI, L))),
        mesh=_MESH,
        compiler_params=_SC_PARAMS,
        scratch_types=[
            pltpu.VMEM((400,), jnp.int32),
            pltpu.VMEM((400,), jnp.float32),
            pltpu.VMEM((400, L), jnp.float32),
            pltpu.VMEM((400, L), jnp.float32),
            pltpu.VMEM((400, L), jnp.float32),
            pltpu.VMEM_SHARED((NU, L), jnp.float32),
            pltpu.VMEM_SHARED((NU, L), jnp.float32),
        ],
    )(h, t, uuh, uud, iih, iid, zeros16)


def _gnnside_body(ego_s_u, ego_s_i, ego_u, ego_i,
                  h_hbm, t_hbm, uuh, uut, uud, iih, iit, iid, zeros_hbm,
                  binsU_out, binsI_out, sideU_out, sideI_out,
                  idx_d0, idx_d1, idx_s0, idx_s1, rows0, rows1,
                  data0, data1, sem0, sem1, bins_sh):
    cid = lax.axis_index("c")
    sid = lax.axis_index("s")
    idx_d = [idx_d0, idx_d1]
    idx_s = [idx_s0, idx_s1]
    rows = [rows0, rows1]
    data = [data0, data1]
    sems = [sem0, sem1]

    _zero_bins(zeros_hbm, bins_sh, sid)
    plsc.subcore_barrier()

    def runA(dst_hbm, src_hbm, table):
        _pipe_gather_scatter(dst_hbm, src_hbm, table, bins_sh,
                             idx_d, idx_s, rows, sems,
                             sid * 50000, 125, 400)

    @pl.when(cid == 0)
    def _():
        runA(h_hbm, t_hbm, ego_s_i)

    @pl.when(cid == 1)
    def _():
        runA(t_hbm, h_hbm, ego_s_u)

    plsc.subcore_barrier()

    @pl.when(cid == 0)
    def _():
        _flush_bins(bins_sh, binsU_out, sid)

    @pl.when(cid == 1)
    def _():
        _flush_bins(bins_sh, binsI_out, sid)

    _zero_bins(zeros_hbm, bins_sh, sid)
    plsc.subcore_barrier()

    def scale_rows(rbuf, dbuf):
        def scale(g, _):
            d16 = dbuf[pl.ds(g * L, L)]
            for j in range(L):
                r = g * L + j
                rbuf[r, pl.ds(0, L)] = rbuf[r, pl.ds(0, L)] * d16[j]
                rbuf[r, pl.ds(L, L)] = rbuf[r, pl.ds(L, L)] * d16[j]
            return 0
        lax.fori_loop(0, 400 // L, scale, 0)

    def runB(dst_hbm, src_hbm, dat_hbm, table):
        nch = jnp.where(sid < 15, 65, 25)
        _pipe_gather_scatter(dst_hbm, src_hbm, table, bins_sh,
                             idx_d, idx_s, rows, sems,
                             sid * 26000, nch, 400,
                             dat_hbm=dat_hbm, data_v=data,
                             scale_fn=scale_rows)

    @pl.when(cid == 0)
    def _():
        runB(uuh, uut, uud, ego_u)

    @pl.when(cid == 1)
    def _():
        runB(iih, iit, iid, ego_i)

    plsc.subcore_barrier()

    @pl.when(cid == 0)
    def _():
        _flush_bins(bins_sh, sideU_out, sid)

    @pl.when(cid == 1)
    def _():
        _flush_bins(bins_sh, sideI_out, sid)


def _gnnside_sc(ego_s_u, ego_s_i, ego_u, ego_i,
                h, t, uuh, uut, uud, iih, iit, iid, zeros32):
    return pl.kernel(
        _gnnside_body,
        out_type=(_sds((NU, D)), _sds((NI, D)), _sds((NU, D)), _sds((NI, D))),
        mesh=_MESH,
        compiler_params=_SC_PARAMS,
        scratch_types=[
            pltpu.VMEM((400,), jnp.int32),
            pltpu.VMEM((400,), jnp.int32),
            pltpu.VMEM((400,), jnp.int32),
            pltpu.VMEM((400,), jnp.int32),
            pltpu.VMEM((400, D), jnp.float32),
            pltpu.VMEM((400, D), jnp.float32),
            pltpu.VMEM((400,), jnp.float32),
            pltpu.VMEM((400,), jnp.float32),
            pltpu.SemaphoreType.DMA,
            pltpu.SemaphoreType.DMA,
            pltpu.VMEM_SHARED((NU, D), jnp.float32),
        ],
    )(ego_s_u, ego_s_i, ego_u, ego_i, h, t, uuh, uut, uud, iih, iit, iid,
      zeros32)


def _pmul_body(gnn_u, gnn_i, h_hbm, t_hbm, p_out,
               idx_h, idx_t, rh0, rh1, rt0, rt1, semh0, semh1, semt0, semt1):
    cid = lax.axis_index("c")
    sid = lax.axis_index("s")
    rows_h = [rh0, rh1]
    rows_t = [rt0, rt1]
    sems_h = [semh0, semh1]
    sems_t = [semt0, semt1]
    base0 = cid * 400000 + sid * 25000
    C = 1000

    def loadidx(k):
        base = base0 + k * C
        pltpu.sync_copy(h_hbm.at[pl.ds(base, C)], idx_h)
        pltpu.sync_copy(t_hbm.at[pl.ds(base, C)], idx_t)

    def startg(b):
        pltpu.async_copy(gnn_u.at[idx_h], rows_h[b], sems_h[b])
        pltpu.async_copy(gnn_i.at[idx_t], rows_t[b], sems_t[b])

    def waitg(b):
        pltpu.make_async_copy(gnn_u.at[idx_h], rows_h[b], sems_h[b]).wait()
        pltpu.make_async_copy(gnn_i.at[idx_t], rows_t[b], sems_t[b]).wait()

    def muland(b, k):
        def mul(r, _):
            rows_h[b][r, :] = rows_h[b][r, :] * rows_t[b][r, :]
            return 0
        lax.fori_loop(0, C, mul, 0)
        pltpu.sync_copy(rows_h[b], p_out.at[pl.ds(base0 + k * C, C), :])

    loadidx(0)
    startg(0)

    def body(m, _):
        k1 = 2 * m + 1
        waitg(0)
        loadidx(k1)
        startg(1)
        muland(0, k1 - 1)
        waitg(1)
        loadidx(k1 + 1)
        startg(0)
        muland(1, k1)
        return 0
    lax.fori_loop(0, 12, body, 0)
    waitg(0)
    muland(0, 24)


def _pmul_sc(gnnb_u, gnnb_i, h, t):
    return pl.kernel(
        _pmul_body,
        out_type=_sds((E, D)),
        mesh=_MESH,
        compiler_params=_SC_PARAMS,
        scratch_types=[
            pltpu.VMEM((1000,), jnp.int32),
            pltpu.VMEM((1000,), jnp.int32),
            pltpu.VMEM((1000, D), jnp.float32),
            pltpu.VMEM((1000, D), jnp.float32),
            pltpu.VMEM((1000, D), jnp.float32),
            pltpu.VMEM((1000, D), jnp.float32),
            pltpu.SemaphoreType.DMA,
            pltpu.SemaphoreType.DMA,
            pltpu.SemaphoreType.DMA,
            pltpu.SemaphoreType.DMA,
        ],
    )(gnnb_u, gnnb_i, h, t)


def _sd_scatter_body(sd_hbm, h_hbm, t_hbm, zeros_hbm,
                     sU_out, sI_out,
                     idx_d0, idx_d1, rows0, rows1, sem0, sem1, bins_sh):
    cid = lax.axis_index("c")
    sid = lax.axis_index("s")
    idx_d = [idx_d0, idx_d1]
    rows = [rows0, rows1]
    sems = [sem0, sem1]
    _zero_bins(zeros_hbm, bins_sh, sid)
    plsc.subcore_barrier()
    C = 2000

    def run(dst_hbm):
        def loadc(k, b):
            base = sid * 50000 + k * C
            pltpu.sync_copy(dst_hbm.at[pl.ds(base, C)], idx_d[b])
            pltpu.async_copy(sd_hbm.at[pl.ds(base, C), :], rows[b], sems[b])

        def waitc(k, b):
            base = sid * 50000 + k * C
            pltpu.make_async_copy(sd_hbm.at[pl.ds(base, C), :], rows[b],
                                  sems[b]).wait()

        def scat(b):
            pltpu.sync_copy(rows[b], bins_sh.at[idx_d[b]], add=True)

        loadc(0, 0)

        def body(m, _):
            k1 = 2 * m + 1
            loadc(k1, 1)
            waitc(k1 - 1, 0)
            scat(0)
            loadc(k1 + 1, 0)
            waitc(k1, 1)
            scat(1)
            return 0
        lax.fori_loop(0, 12, body, 0)
        waitc(24, 0)
        scat(0)

    @pl.when(cid == 0)
    def _():
        run(h_hbm)

    @pl.when(cid == 1)
    def _():
        run(t_hbm)

    plsc.subcore_barrier()

    @pl.when(cid == 0)
    def _():
        _flush_bins(bins_sh, sU_out, sid)

    @pl.when(cid == 1)
    def _():
        _flush_bins(bins_sh, sI_out, sid)


def _sd_scatter_sc(sd, h, t, zeros16):
    return pl.kernel(
        _sd_scatter_body,
        out_type=(_sds((NU, K)), _sds((NI, K))),
        mesh=_MESH,
        compiler_params=_SC_PARAMS,
        scratch_types=[
            pltpu.VMEM((2000,), jnp.int32),
            pltpu.VMEM((2000,), jnp.int32),
            pltpu.VMEM((2000, K), jnp.float32),
            pltpu.VMEM((2000, K), jnp.float32),
            pltpu.SemaphoreType.DMA,
            pltpu.SemaphoreType.DMA,
            pltpu.VMEM_SHARED((NU, K), jnp.float32),
        ],
    )(sd, h, t, zeros16)


_TBLK = 2000


def _scale_body(x_ref, v_ref, o_ref):
    o_ref[...] = x_ref[...] * v_ref[...]


def _scale2_body(xu_ref, xi_ref, vu_ref, vi_ref, ou_ref, oi_ref):
    ou_ref[...] = xu_ref[...] * vu_ref[...]
    oi_ref[...] = xi_ref[...] * vi_ref[...]


def _scale2_tc(xu, xi, vu2, vi2):
    blkd = pl.BlockSpec((_TBLK, D), lambda i: (i, 0))
    blk1 = pl.BlockSpec((_TBLK, 1), lambda i: (i, 0))
    return pl.pallas_call(
        _scale2_body,
        grid=(NU // _TBLK,),
        in_specs=[blkd, blkd, blk1, blk1],
        out_specs=[blkd, blkd],
        out_shape=(_sds((NU, D)), _sds((NI, D))),
    )(xu, xi, vu2, vi2)


def _scale_tc(x, v2d):
    n = x.shape[0]
    return pl.pallas_call(
        _scale_body,
        grid=(n // _TBLK,),
        in_specs=[pl.BlockSpec((_TBLK, x.shape[1]), lambda i: (i, 0)),
                  pl.BlockSpec((_TBLK, 1), lambda i: (i, 0))],
        out_specs=pl.BlockSpec((_TBLK, x.shape[1]), lambda i: (i, 0)),
        out_shape=_sds((n, x.shape[1])),
    )(x, v2d)


def _intent_body(p_ref, w_ref, o_ref):
    p = p_ref[...]
    logits = jnp.dot(p, w_ref[...], preferred_element_type=jnp.float32)
    m = jnp.max(logits, axis=1, keepdims=True)
    e = jnp.exp(logits - m)
    dist = e / jnp.sum(e, axis=1, keepdims=True)
    alpha = (jnp.sum(p, axis=1, keepdims=True) + 1.0) * 0.5
    o_ref[...] = alpha * dist


def _intent_tc(p, w):
    blk = 8000
    return pl.pallas_call(
        _intent_body,
        grid=(E // blk,),
        in_specs=[pl.BlockSpec((blk, D), lambda i: (i, 0)),
                  pl.BlockSpec((D, K), lambda i: (0, 0))],
        out_specs=pl.BlockSpec((blk, K), lambda i: (i, 0)),
        out_shape=_sds((E, K)),
    )(p, w)


def _combine_body(gnn_ref, side_ref, s_ref, w_ref, inv_ref, v_ref, ae_ref,
                  ego_ref, egos_ref, aeo_ref):
    intent = lax.dot_general(s_ref[...], w_ref[...],
                             (((1,), (1,)), ((), ())),
                             preferred_element_type=jnp.float32)
    ego = gnn_ref[...] + inv_ref[...] * side_ref[...] + intent
    ego_ref[...] = ego
    egos_ref[...] = v_ref[...] * ego
    aeo_ref[...] = ae_ref[...] + ego


def _combine2_body(gu_ref, su_ref, squ_ref, w_ref, iu_ref, vu_ref, au_ref,
                   gi_ref, si_ref, sqi_ref, ii_ref, vi_ref, ai_ref,
                   egou_ref, egosu_ref, aeu_ref,
                   egoi_ref, egosi_ref, aei_ref):
    dn = (((1,), (1,)), ((), ()))
    intu = lax.dot_general(squ_ref[...], w_ref[...], dn,
                           preferred_element_type=jnp.float32)
    inti = lax.dot_general(sqi_ref[...], w_ref[...], dn,
                           preferred_element_type=jnp.float32)
    egou = gu_ref[...] + iu_ref[...] * su_ref[...] + intu
    egoi = gi_ref[...] + ii_ref[...] * si_ref[...] + inti
    egou_ref[...] = egou
    egoi_ref[...] = egoi
    egosu_ref[...] = vu_ref[...] * egou
    egosi_ref[...] = vi_ref[...] * egoi
    aeu_ref[...] = au_ref[...] + egou
    aei_ref[...] = ai_ref[...] + egoi


def _combine2_tc(gnn_u, sideraw_u, s_u, w, inv_u2, v_u2, ae_u,
                 gnn_i, sideraw_i, s_i, inv_i2, v_i2, ae_i):
    blkd = pl.BlockSpec((_TBLK, D), lambda i: (i, 0))
    blkk = pl.BlockSpec((_TBLK, K), lambda i: (i, 0))
    blk1 = pl.BlockSpec((_TBLK, 1), lambda i: (i, 0))
    blkw = pl.BlockSpec((D, K), lambda i: (0, 0))
    return pl.pallas_call(
        _combine2_body,
        grid=(NU // _TBLK,),
        in_specs=[blkd, blkd, blkk, blkw, blk1, blk1, blkd,
                  blkd, blkd, blkk, blk1, blk1, blkd],
        out_specs=[blkd, blkd, blkd, blkd, blkd, blkd],
        out_shape=(_sds((NU, D)), _sds((NU, D)), _sds((NU, D)),
                   _sds((NI, D)), _sds((NI, D)), _sds((NI, D))),
    )(gnn_u, sideraw_u, s_u, w, inv_u2, v_u2, ae_u,
      gnn_i, sideraw_i, s_i, inv_i2, v_i2, ae_i)


def kernel(user_emb, item_emb, intents, uu_data, ii_data,
           h_list, t_list, uu_h, uu_t, ii_h, ii_t):
    zeros32 = jnp.zeros((NU, D), jnp.float32)
    zeros16 = jnp.zeros((NU, K), jnp.float32)

    v_u, v_i, inv_u, inv_i = _precompute_sc(h_list, t_list, uu_h, uu_data,
                                            ii_h, ii_data, zeros16)
    v_u2 = v_u[:, 0:1]
    v_i2 = v_i[:, 0:1]
    inv_u2 = inv_u[:, 0:1]
    inv_i2 = inv_i[:, 0:1]

    ego_u, ego_i = user_emb, item_emb
    ae_u, ae_i = user_emb, item_emb
    ego_s_u = _scale_tc(ego_u, v_u2)
    ego_s_i = _scale_tc(ego_i, v_i2)

    for _ in range(NLAYERS):
        binsA_u, binsA_i, sideraw_u, sideraw_i = _gnnside_sc(
            ego_s_u, ego_s_i, ego_u, ego_i, h_list, t_list,
            uu_h, uu_t, uu_data, ii_h, ii_t, ii_data, zeros32)
        gnn_u, gnn_i = _scale2_tc(binsA_u, binsA_i, v_u2, v_i2)
        p = _pmul_sc(gnn_u, gnn_i, h_list, t_list)
        sd = _intent_tc(p, intents)
        s_u, s_i = _sd_scatter_sc(sd, h_list, t_list, zeros16)
        (ego_u, ego_s_u, ae_u,
         ego_i, ego_s_i, ae_i) = _combine2_tc(
            gnn_u, sideraw_u, s_u, intents, inv_u2, v_u2, ae_u,
            gnn_i, sideraw_i, s_i, inv_i2, v_i2, ae_i)

    return jnp.concatenate([ae_u, ae_i], axis=0)
